# Initial kernel scaffold; baseline (speedup 1.0000x reference)
import jax
import jax.numpy as jnp
from jax.experimental import pallas as pl


def kernel(text, emb_table, W1, b1, W2, b2):
    raise NotImplementedError("write your pallas kernel here")


# speedup vs baseline: 20.5824x; 20.5824x over previous
import jax
import jax.numpy as jnp
from jax import lax
from jax.experimental import pallas as pl
from jax.experimental.pallas import tpu as pltpu
from jax.experimental.pallas import tpu_sc as plsc

VOCAB = 100000
EMB = 128
OUT = 2
HID = (EMB + OUT) // 2
SEQ = 200
BATCH = 4096

NC = 2
NS = 16
L = 16
NW = NC * NS
ROWS_PER_W = BATCH // NW
WORDS_PER_W = ROWS_PER_W * SEQ
FULL_CHUNKS = SEQ // L
TAIL = SEQ - FULL_CHUNKS * L
CHUNK_ROWS = 32
NCHUNKS = ROWS_PER_W // CHUNK_ROWS
CHUNK_WORDS = CHUNK_ROWS * SEQ

ROW_BLK = 1024
GRID = (VOCAB + ROW_BLK - 1) // ROW_BLK


def _project_body(emb_ref, w1_ref, w2_ref, p_ref):
    wc = lax.dot_general(w2_ref[...], w1_ref[...],
                         (((1,), (0,)), ((), ())),
                         preferred_element_type=jnp.float32)
    p_ref[...] = lax.dot_general(wc, emb_ref[...],
                                 (((1,), (1,)), ((), ())),
                                 preferred_element_type=jnp.float32)


def _pool_body(text_hbm, p_hbm, bias_hbm, out_hbm, text_v, pvals, rowacc,
               out_v, bias_v):
    c = lax.axis_index("c")
    s = lax.axis_index("s")
    wid = s * NC + c
    wbase = wid * WORDS_PER_W
    tail_mask = lax.iota(jnp.int32, L) < TAIL
    lanes = lax.iota(jnp.int32, L)
    pltpu.sync_copy(bias_hbm, bias_v)

    for j in range(OUT):
        pltpu.sync_copy(p_hbm.at[j], pvals)

        for cb in range(NCHUNKS):
            pltpu.sync_copy(
                text_hbm.at[pl.ds(wbase + cb * CHUNK_WORDS, CHUNK_WORDS)],
                text_v.at[pl.ds(0, CHUNK_WORDS)])
            text_v[pl.ds(CHUNK_WORDS, L)] = jnp.zeros((L,), jnp.int32)

            def row_body(r, carry):
                rbase = r * SEQ
                acc = jnp.zeros((L,), jnp.float32)
                for ck in range(FULL_CHUNKS):
                    idx = text_v[pl.ds(rbase + ck * L, L)]
                    acc = acc + plsc.load_gather(pvals, [idx])
                tidx = text_v[pl.ds(rbase + FULL_CHUNKS * L, L)]
                tv = plsc.load_gather(pvals, [tidx])
                acc = acc + jnp.where(tail_mask, tv, 0.0)
                rowacc[pl.ds(r * L, L)] = acc
                return carry

            lax.fori_loop(0, CHUNK_ROWS, row_body, 0)

            def grp_body(g, carry, j=j, cb=cb):
                base = g * L * L
                tot = jnp.zeros((L,), jnp.float32)
                for l in range(L):
                    tot = tot + plsc.load_gather(rowacc, [base + lanes * L + l])
                plsc.store_scatter(
                    out_v,
                    [cb * CHUNK_ROWS + g * L + lanes,
                     jnp.full((L,), j, jnp.int32)],
                    tot * (1.0 / SEQ) + bias_v[pl.ds(j * L, L)])
                return carry

            lax.fori_loop(0, CHUNK_ROWS // L, grp_body, 0)

    rowbase = wid * ROWS_PER_W
    pltpu.sync_copy(out_v, out_hbm.at[pl.ds(rowbase, ROWS_PER_W), :])


_pool = pl.kernel(
    _pool_body,
    out_type=jax.ShapeDtypeStruct((BATCH, OUT), jnp.float32),
    mesh=plsc.VectorSubcoreMesh(core_axis_name="c", subcore_axis_name="s",
                                num_cores=NC, num_subcores=NS),
    compiler_params=pltpu.CompilerParams(needs_layout_passes=False),
    scratch_types=[
        pltpu.VMEM((CHUNK_WORDS + L,), jnp.int32),
        pltpu.VMEM((VOCAB,), jnp.float32),
        pltpu.VMEM((CHUNK_ROWS * L,), jnp.float32),
        pltpu.VMEM((ROWS_PER_W, OUT), jnp.float32),
        pltpu.VMEM((OUT * L,), jnp.float32),
    ],
)


def kernel(text, emb_table, W1, b1, W2, b2):
    p = pl.pallas_call(
        _project_body,
        grid=(GRID,),
        in_specs=[
            pl.BlockSpec((ROW_BLK, EMB), lambda i: (i, 0)),
            pl.BlockSpec((HID, EMB), lambda i: (0, 0)),
            pl.BlockSpec((OUT, HID), lambda i: (0, 0)),
        ],
        out_specs=pl.BlockSpec((OUT, ROW_BLK), lambda i: (0, i)),
        out_shape=jax.ShapeDtypeStruct((OUT, VOCAB), jnp.float32),
    )(emb_table, W1, W2)
    bias = W2 @ b1 + b2
    bias_rep = jnp.broadcast_to(bias[:, None], (OUT, L)).reshape(OUT * L)
    return _pool(text.reshape(-1), p, bias_rep)


# speedup vs baseline: 28.9387x; 1.4060x over previous
import jax
import jax.numpy as jnp
from jax import lax
from jax.experimental import pallas as pl
from jax.experimental.pallas import tpu as pltpu
from jax.experimental.pallas import tpu_sc as plsc

VOCAB = 100000
EMB = 128
OUT = 2
HID = (EMB + OUT) // 2
SEQ = 200
BATCH = 4096

NC = 2
NS = 16
L = 16
NW = NC * NS
ROWS_PER_W = BATCH // NW
WORDS_PER_W = ROWS_PER_W * SEQ
FULL_CHUNKS = SEQ // L
TAIL = SEQ - FULL_CHUNKS * L
CHUNK_ROWS = 32
NCHUNKS = ROWS_PER_W // CHUNK_ROWS
CHUNK_WORDS = CHUNK_ROWS * SEQ

ROW_BLK = 4096
GRID = (VOCAB + ROW_BLK - 1) // ROW_BLK


def _project_body(emb_ref, w1_ref, w2_ref, p_ref):
    wc = lax.dot_general(w2_ref[...], w1_ref[...],
                         (((1,), (0,)), ((), ())),
                         preferred_element_type=jnp.float32)
    p_ref[...] = lax.dot_general(wc, emb_ref[...],
                                 (((1,), (1,)), ((), ())),
                                 preferred_element_type=jnp.float32)


def _pool_body(text_hbm, p_hbm, bias_hbm, out_hbm, text_v, pvals, rowacc,
               out_v, bias_v):
    c = lax.axis_index("c")
    s = lax.axis_index("s")
    wid = s * NC + c
    wbase = wid * WORDS_PER_W
    tail_mask = lax.iota(jnp.int32, L) < TAIL
    lanes = lax.iota(jnp.int32, L)
    pltpu.sync_copy(bias_hbm, bias_v)

    for j in range(OUT):
        pltpu.sync_copy(p_hbm.at[j], pvals)

        for cb in range(NCHUNKS):
            pltpu.sync_copy(
                text_hbm.at[pl.ds(wbase + cb * CHUNK_WORDS, CHUNK_WORDS)],
                text_v.at[pl.ds(0, CHUNK_WORDS)])
            text_v[pl.ds(CHUNK_WORDS, L)] = jnp.zeros((L,), jnp.int32)

            def row_body(r, carry):
                rbase = r * SEQ
                acc = jnp.zeros((L,), jnp.float32)
                for ck in range(FULL_CHUNKS):
                    idx = text_v[pl.ds(rbase + ck * L, L)]
                    acc = acc + plsc.load_gather(pvals, [idx])
                tidx = text_v[pl.ds(rbase + FULL_CHUNKS * L, L)]
                tv = plsc.load_gather(pvals, [tidx])
                acc = acc + jnp.where(tail_mask, tv, 0.0)
                rowacc[pl.ds(r * L, L)] = acc
                return carry

            lax.fori_loop(0, CHUNK_ROWS, row_body, 0)

            def grp_body(g, carry, j=j, cb=cb):
                base = g * L * L
                tot = jnp.zeros((L,), jnp.float32)
                for l in range(L):
                    tot = tot + plsc.load_gather(rowacc, [base + lanes * L + l])
                plsc.store_scatter(
                    out_v,
                    [cb * CHUNK_ROWS + g * L + lanes,
                     jnp.full((L,), j, jnp.int32)],
                    tot * (1.0 / SEQ) + bias_v[pl.ds(j * L, L)])
                return carry

            lax.fori_loop(0, CHUNK_ROWS // L, grp_body, 0)

    rowbase = wid * ROWS_PER_W
    pltpu.sync_copy(out_v, out_hbm.at[pl.ds(rowbase, ROWS_PER_W), :])


_pool = pl.kernel(
    _pool_body,
    out_type=jax.ShapeDtypeStruct((BATCH, OUT), jnp.float32),
    mesh=plsc.VectorSubcoreMesh(core_axis_name="c", subcore_axis_name="s",
                                num_cores=NC, num_subcores=NS),
    compiler_params=pltpu.CompilerParams(needs_layout_passes=False),
    scratch_types=[
        pltpu.VMEM((CHUNK_WORDS + L,), jnp.int32),
        pltpu.VMEM((VOCAB,), jnp.float32),
        pltpu.VMEM((CHUNK_ROWS * L,), jnp.float32),
        pltpu.VMEM((ROWS_PER_W, OUT), jnp.float32),
        pltpu.VMEM((OUT * L,), jnp.float32),
    ],
)


def kernel(text, emb_table, W1, b1, W2, b2):
    p = pl.pallas_call(
        _project_body,
        grid=(GRID,),
        in_specs=[
            pl.BlockSpec((ROW_BLK, EMB), lambda i: (i, 0)),
            pl.BlockSpec((HID, EMB), lambda i: (0, 0)),
            pl.BlockSpec((OUT, HID), lambda i: (0, 0)),
        ],
        out_specs=pl.BlockSpec((OUT, ROW_BLK), lambda i: (0, i)),
        out_shape=jax.ShapeDtypeStruct((OUT, VOCAB), jnp.float32),
    )(emb_table, W1, W2)
    bias = W2 @ b1 + b2
    bias_rep = jnp.broadcast_to(bias[:, None], (OUT, L)).reshape(OUT * L)
    return _pool(text.reshape(-1), p, bias_rep)


# speedup vs baseline: 35.5434x; 1.2282x over previous
import jax
import jax.numpy as jnp
from jax import lax
from jax.experimental import pallas as pl
from jax.experimental.pallas import tpu as pltpu
from jax.experimental.pallas import tpu_sc as plsc

VOCAB = 100000
EMB = 128
OUT = 2
HID = (EMB + OUT) // 2
SEQ = 200
BATCH = 4096

NC = 2
NS = 16
L = 16
NW = NC * NS
ROWS_PER_W = BATCH // NW
WORDS_PER_W = ROWS_PER_W * SEQ
FULL_CHUNKS = SEQ // L
TAIL = SEQ - FULL_CHUNKS * L
CHUNK_ROWS = 32
NCHUNKS = ROWS_PER_W // CHUNK_ROWS
CHUNK_WORDS = CHUNK_ROWS * SEQ

ROW_BLK = 4096
GRID = (VOCAB + ROW_BLK - 1) // ROW_BLK


def _project_body(emb_ref, w1_ref, w2_ref, p_ref):
    wc = lax.dot_general(w2_ref[...], w1_ref[...],
                         (((1,), (0,)), ((), ())),
                         preferred_element_type=jnp.float32)
    pblk = lax.dot_general(wc, emb_ref[...],
                           (((1,), (1,)), ((), ())),
                           preferred_element_type=jnp.float32)
    lo = lax.bitcast_convert_type(
        lax.convert_element_type(pblk[0:1, :], jnp.bfloat16), jnp.uint16)
    hi = lax.bitcast_convert_type(
        lax.convert_element_type(pblk[1:2, :], jnp.bfloat16), jnp.uint16)
    packed = lo.astype(jnp.uint32) | (hi.astype(jnp.uint32) << 16)
    p_ref[...] = lax.bitcast_convert_type(packed, jnp.int32)


def _unpack_pair(word):
    v0 = plsc.bitcast(lax.shift_left(word, 16), jnp.float32)
    v1 = plsc.bitcast(jnp.bitwise_and(word, jnp.int32(-65536)), jnp.float32)
    return v0, v1


def _pool_body(text_hbm, p_hbm, bias_hbm, out_hbm, text_v, pvals, rowacc0,
               rowacc1, out_v, bias_v):
    c = lax.axis_index("c")
    s = lax.axis_index("s")
    wid = s * NC + c
    wbase = wid * WORDS_PER_W
    tail_mask = lax.iota(jnp.int32, L) < TAIL
    lanes = lax.iota(jnp.int32, L)
    pltpu.sync_copy(bias_hbm, bias_v)
    pltpu.sync_copy(p_hbm.at[0], pvals)

    for cb in range(NCHUNKS):
        pltpu.sync_copy(
            text_hbm.at[pl.ds(wbase + cb * CHUNK_WORDS, CHUNK_WORDS)],
            text_v.at[pl.ds(0, CHUNK_WORDS)])
        text_v[pl.ds(CHUNK_WORDS, L)] = jnp.zeros((L,), jnp.int32)

        @plsc.parallel_loop(0, CHUNK_ROWS, unroll=4)
        def row_body(r):
            rbase = r * SEQ
            acc0 = jnp.zeros((L,), jnp.float32)
            acc1 = jnp.zeros((L,), jnp.float32)
            for ck in range(FULL_CHUNKS):
                idx = text_v[pl.ds(rbase + ck * L, L)]
                v0, v1 = _unpack_pair(plsc.load_gather(pvals, [idx]))
                acc0 = acc0 + v0
                acc1 = acc1 + v1
            tidx = text_v[pl.ds(rbase + FULL_CHUNKS * L, L)]
            tv0, tv1 = _unpack_pair(plsc.load_gather(pvals, [tidx]))
            acc0 = acc0 + jnp.where(tail_mask, tv0, 0.0)
            acc1 = acc1 + jnp.where(tail_mask, tv1, 0.0)
            rowacc0[pl.ds(r * L, L)] = acc0
            rowacc1[pl.ds(r * L, L)] = acc1

        @plsc.parallel_loop(0, CHUNK_ROWS // L, unroll=2)
        def grp_body(g, cb=cb):
            base = g * L * L
            tot0 = jnp.zeros((L,), jnp.float32)
            tot1 = jnp.zeros((L,), jnp.float32)
            for l in range(L):
                tot0 = tot0 + plsc.load_gather(rowacc0, [base + lanes * L + l])
                tot1 = tot1 + plsc.load_gather(rowacc1, [base + lanes * L + l])
            rows = cb * CHUNK_ROWS + g * L + lanes
            plsc.store_scatter(
                out_v, [rows, jnp.zeros((L,), jnp.int32)],
                tot0 * (1.0 / SEQ) + bias_v[pl.ds(0, L)])
            plsc.store_scatter(
                out_v, [rows, jnp.full((L,), 1, jnp.int32)],
                tot1 * (1.0 / SEQ) + bias_v[pl.ds(L, L)])

    rowbase = wid * ROWS_PER_W
    pltpu.sync_copy(out_v, out_hbm.at[pl.ds(rowbase, ROWS_PER_W), :])


_pool = pl.kernel(
    _pool_body,
    out_type=jax.ShapeDtypeStruct((BATCH, OUT), jnp.float32),
    mesh=plsc.VectorSubcoreMesh(core_axis_name="c", subcore_axis_name="s",
                                num_cores=NC, num_subcores=NS),
    compiler_params=pltpu.CompilerParams(needs_layout_passes=False),
    scratch_types=[
        pltpu.VMEM((CHUNK_WORDS + L,), jnp.int32),
        pltpu.VMEM((VOCAB,), jnp.int32),
        pltpu.VMEM((CHUNK_ROWS * L,), jnp.float32),
        pltpu.VMEM((CHUNK_ROWS * L,), jnp.float32),
        pltpu.VMEM((ROWS_PER_W, OUT), jnp.float32),
        pltpu.VMEM((OUT * L,), jnp.float32),
    ],
)


def kernel(text, emb_table, W1, b1, W2, b2):
    p = pl.pallas_call(
        _project_body,
        grid=(GRID,),
        in_specs=[
            pl.BlockSpec((ROW_BLK, EMB), lambda i: (i, 0)),
            pl.BlockSpec((HID, EMB), lambda i: (0, 0)),
            pl.BlockSpec((OUT, HID), lambda i: (0, 0)),
        ],
        out_specs=pl.BlockSpec((1, ROW_BLK), lambda i: (0, i)),
        out_shape=jax.ShapeDtypeStruct((1, VOCAB), jnp.int32),
    )(emb_table, W1, W2)
    bias = W2 @ b1 + b2
    bias_rep = jnp.broadcast_to(bias[:, None], (OUT, L)).reshape(OUT * L)
    return _pool(text.reshape(-1), p, bias_rep)


# speedup vs baseline: 39.2604x; 1.1046x over previous
import jax
import jax.numpy as jnp
from jax import lax
from jax.experimental import pallas as pl
from jax.experimental.pallas import tpu as pltpu
from jax.experimental.pallas import tpu_sc as plsc

VOCAB = 100000
EMB = 128
OUT = 2
HID = (EMB + OUT) // 2
SEQ = 200
BATCH = 4096

NC = 2
NS = 16
L = 16
NW = NC * NS
ROWS_PER_W = BATCH // NW
WORDS_PER_W = ROWS_PER_W * SEQ
FULL_CHUNKS = SEQ // L
TAIL = SEQ - FULL_CHUNKS * L
CHUNK_ROWS = 32
NCHUNKS = ROWS_PER_W // CHUNK_ROWS
CHUNK_WORDS = CHUNK_ROWS * SEQ

ROW_BLK = 8192
GRID = (VOCAB + ROW_BLK - 1) // ROW_BLK


def _project_body(emb_ref, w1_ref, w2_ref, p_ref):
    wc = lax.dot_general(w2_ref[...], w1_ref[...],
                         (((1,), (0,)), ((), ())),
                         preferred_element_type=jnp.float32)
    pblk = lax.dot_general(wc, emb_ref[...],
                           (((1,), (1,)), ((), ())),
                           preferred_element_type=jnp.float32)
    lo = lax.bitcast_convert_type(
        lax.convert_element_type(pblk[0:1, :], jnp.bfloat16), jnp.uint16)
    hi = lax.bitcast_convert_type(
        lax.convert_element_type(pblk[1:2, :], jnp.bfloat16), jnp.uint16)
    packed = lo.astype(jnp.uint32) | (hi.astype(jnp.uint32) << 16)
    p_ref[...] = lax.bitcast_convert_type(packed, jnp.int32)


def _unpack_pair(word):
    v0 = plsc.bitcast(lax.shift_left(word, 16), jnp.float32)
    v1 = plsc.bitcast(jnp.bitwise_and(word, jnp.int32(-65536)), jnp.float32)
    return v0, v1


def _pool_body(text_hbm, p_hbm, bias_hbm, out_hbm, text_v, pvals, rowacc0,
               rowacc1, out_v, bias_v):
    c = lax.axis_index("c")
    s = lax.axis_index("s")
    wid = s * NC + c
    wbase = wid * WORDS_PER_W
    tail_mask = lax.iota(jnp.int32, L) < TAIL
    lanes = lax.iota(jnp.int32, L)
    pltpu.sync_copy(bias_hbm, bias_v)
    pltpu.sync_copy(p_hbm.at[0], pvals)

    for cb in range(NCHUNKS):
        pltpu.sync_copy(
            text_hbm.at[pl.ds(wbase + cb * CHUNK_WORDS, CHUNK_WORDS)],
            text_v.at[pl.ds(0, CHUNK_WORDS)])
        text_v[pl.ds(CHUNK_WORDS, L)] = jnp.zeros((L,), jnp.int32)

        @plsc.parallel_loop(0, CHUNK_ROWS, unroll=4)
        def row_body(r):
            rbase = r * SEQ
            acc0 = jnp.zeros((L,), jnp.float32)
            acc1 = jnp.zeros((L,), jnp.float32)
            for ck in range(FULL_CHUNKS):
                idx = text_v[pl.ds(rbase + ck * L, L)]
                v0, v1 = _unpack_pair(plsc.load_gather(pvals, [idx]))
                acc0 = acc0 + v0
                acc1 = acc1 + v1
            tidx = text_v[pl.ds(rbase + FULL_CHUNKS * L, L)]
            tv0, tv1 = _unpack_pair(plsc.load_gather(pvals, [tidx]))
            acc0 = acc0 + jnp.where(tail_mask, tv0, 0.0)
            acc1 = acc1 + jnp.where(tail_mask, tv1, 0.0)
            rowacc0[pl.ds(r * L, L)] = acc0
            rowacc1[pl.ds(r * L, L)] = acc1

        @plsc.parallel_loop(0, CHUNK_ROWS // L, unroll=2)
        def grp_body(g, cb=cb):
            base = g * L * L
            tot0 = jnp.zeros((L,), jnp.float32)
            tot1 = jnp.zeros((L,), jnp.float32)
            for l in range(L):
                tot0 = tot0 + plsc.load_gather(rowacc0, [base + lanes * L + l])
                tot1 = tot1 + plsc.load_gather(rowacc1, [base + lanes * L + l])
            rows = cb * CHUNK_ROWS + g * L + lanes
            plsc.store_scatter(
                out_v, [rows, jnp.zeros((L,), jnp.int32)],
                tot0 * (1.0 / SEQ) + bias_v[pl.ds(0, L)])
            plsc.store_scatter(
                out_v, [rows, jnp.full((L,), 1, jnp.int32)],
                tot1 * (1.0 / SEQ) + bias_v[pl.ds(L, L)])

    rowbase = wid * ROWS_PER_W
    pltpu.sync_copy(out_v, out_hbm.at[pl.ds(rowbase, ROWS_PER_W), :])


_pool = pl.kernel(
    _pool_body,
    out_type=jax.ShapeDtypeStruct((BATCH, OUT), jnp.float32),
    mesh=plsc.VectorSubcoreMesh(core_axis_name="c", subcore_axis_name="s",
                                num_cores=NC, num_subcores=NS),
    compiler_params=pltpu.CompilerParams(needs_layout_passes=False),
    scratch_types=[
        pltpu.VMEM((CHUNK_WORDS + L,), jnp.int32),
        pltpu.VMEM((VOCAB,), jnp.int32),
        pltpu.VMEM((CHUNK_ROWS * L,), jnp.float32),
        pltpu.VMEM((CHUNK_ROWS * L,), jnp.float32),
        pltpu.VMEM((ROWS_PER_W, OUT), jnp.float32),
        pltpu.VMEM((OUT * L,), jnp.float32),
    ],
)


def kernel(text, emb_table, W1, b1, W2, b2):
    p = pl.pallas_call(
        _project_body,
        grid=(GRID,),
        in_specs=[
            pl.BlockSpec((ROW_BLK, EMB), lambda i: (i, 0)),
            pl.BlockSpec((HID, EMB), lambda i: (0, 0)),
            pl.BlockSpec((OUT, HID), lambda i: (0, 0)),
        ],
        out_specs=pl.BlockSpec((1, ROW_BLK), lambda i: (0, i)),
        out_shape=jax.ShapeDtypeStruct((1, VOCAB), jnp.int32),
    )(emb_table, W1, W2)
    bias = W2 @ b1 + b2
    bias_rep = jnp.broadcast_to(bias[:, None], (OUT, L)).reshape(OUT * L)
    return _pool(text.reshape(-1), p, bias_rep)
